# baseline (speedup 1.0000x reference)
import functools

import jax
import jax.numpy as jnp
from jax import lax
from jax.experimental import pallas as pl
from jax.experimental.pallas import tpu as pltpu

EMB = 64
NUM_CODES = 1024
COMMIT = 0.25
TOKEN_BLOCK = 2048


def _vq_block(x_ref, cb_ref, out_ref, loss_ref):
    i = pl.program_id(0)
    xb = x_ref[...]
    cb = cb_ref[...]

    xc = lax.dot_general(
        xb, cb, (((1,), (1,)), ((), ())),
        preferred_element_type=jnp.float32,
        precision=lax.Precision.HIGHEST)
    x2 = jnp.sum(xb * xb, axis=1, keepdims=True)
    w2 = jnp.sum(cb * cb, axis=1)[None, :]
    dist = (x2 + w2) - 2.0 * xc

    mind = jnp.min(dist, axis=1, keepdims=True)
    iota = lax.broadcasted_iota(jnp.int32, dist.shape, 1)
    idx = jnp.min(jnp.where(dist == mind, iota, NUM_CODES),
                  axis=1, keepdims=True)

    s = mind - dist
    lse = jnp.log(jnp.sum(jnp.exp(s), axis=1, keepdims=True))
    m = jnp.where(iota == idx, s - lse + 1.0, s - lse) * 0.5

    out_ref[...] = lax.dot_general(
        m, cb, (((1,), (0,)), ((), ())),
        preferred_element_type=jnp.float32,
        precision=lax.Precision.HIGHEST)

    part = jnp.sum(mind, axis=(0, 1), keepdims=True)

    @pl.when(i == 0)
    def _init():
        loss_ref[...] = jnp.zeros_like(loss_ref)

    loss_ref[...] += part


@jax.jit
def kernel(x, codebook):
    n = x.shape[0] * x.shape[1]
    flat_x = x.reshape(n, EMB)
    grid = n // TOKEN_BLOCK

    out, loss = pl.pallas_call(
        _vq_block,
        grid=(grid,),
        in_specs=[
            pl.BlockSpec((TOKEN_BLOCK, EMB), lambda i: (i, 0)),
            pl.BlockSpec((NUM_CODES, EMB), lambda i: (0, 0)),
        ],
        out_specs=[
            pl.BlockSpec((TOKEN_BLOCK, EMB), lambda i: (i, 0)),
            pl.BlockSpec((1, 1), lambda i: (0, 0)),
        ],
        out_shape=[
            jax.ShapeDtypeStruct((n, EMB), jnp.float32),
            jax.ShapeDtypeStruct((1, 1), jnp.float32),
        ],
    )(flat_x, codebook)

    vq_loss = loss[0, 0] * ((1.0 + COMMIT) / (n * EMB))
    return (out.reshape(x.shape), vq_loss)


# speedup vs baseline: 3.3743x; 3.3743x over previous
import functools

import jax
import jax.numpy as jnp
from jax import lax
from jax.experimental import pallas as pl
from jax.experimental.pallas import tpu as pltpu

EMB = 64
NUM_CODES = 1024
COMMIT = 0.25
TOKEN_BLOCK = 2048


def _vq_block(x_ref, cb_ref, out_ref, loss_ref):
    i = pl.program_id(0)
    xb = x_ref[...]
    cb = cb_ref[...]

    w2 = jnp.sum(cb * cb, axis=1)[None, :]
    sim = lax.dot_general(
        xb + xb, cb, (((1,), (1,)), ((), ())),
        preferred_element_type=jnp.float32) - w2

    maxs = jnp.max(sim, axis=1, keepdims=True)
    iota = lax.broadcasted_iota(jnp.int32, sim.shape, 1)
    idx = jnp.min(jnp.where(sim == maxs, iota, NUM_CODES),
                  axis=1, keepdims=True)

    es = jnp.exp(sim - maxs)
    lse = jnp.log(jnp.sum(es, axis=1, keepdims=True))
    m = (sim - (maxs + lse)) + (iota == idx).astype(jnp.float32)

    out_ref[...] = lax.dot_general(
        m, cb * 0.5, (((1,), (0,)), ((), ())),
        preferred_element_type=jnp.float32)

    x2 = jnp.sum(xb * xb, axis=1, keepdims=True)
    part = jnp.sum(x2 - maxs, axis=(0, 1), keepdims=True)

    @pl.when(i == 0)
    def _init():
        loss_ref[...] = jnp.zeros_like(loss_ref)

    loss_ref[...] += part


@jax.jit
def kernel(x, codebook):
    n = x.shape[0] * x.shape[1]
    flat_x = x.reshape(n, EMB)
    grid = n // TOKEN_BLOCK

    out, loss = pl.pallas_call(
        _vq_block,
        grid=(grid,),
        in_specs=[
            pl.BlockSpec((TOKEN_BLOCK, EMB), lambda i: (i, 0)),
            pl.BlockSpec((NUM_CODES, EMB), lambda i: (0, 0)),
        ],
        out_specs=[
            pl.BlockSpec((TOKEN_BLOCK, EMB), lambda i: (i, 0)),
            pl.BlockSpec((1, 1), lambda i: (0, 0)),
        ],
        out_shape=[
            jax.ShapeDtypeStruct((n, EMB), jnp.float32),
            jax.ShapeDtypeStruct((1, 1), jnp.float32),
        ],
    )(flat_x, codebook)

    vq_loss = loss[0, 0] * ((1.0 + COMMIT) / (n * EMB))
    return (out.reshape(x.shape), vq_loss)


# speedup vs baseline: 3.8947x; 1.1542x over previous
import functools

import jax
import jax.numpy as jnp
from jax import lax
from jax.experimental import pallas as pl
from jax.experimental.pallas import tpu as pltpu

EMB = 64
NUM_CODES = 1024
COMMIT = 0.25
TOKEN_BLOCK = 2048


def _vq_block(x_ref, cb_ref, out_ref, loss_ref):
    xb = x_ref[...]
    cb = cb_ref[...]

    w2 = jnp.sum(cb * cb, axis=1)[None, :]
    sim = lax.dot_general(
        xb + xb, cb, (((1,), (1,)), ((), ())),
        preferred_element_type=jnp.float32) - w2

    maxs = jnp.max(sim, axis=1, keepdims=True)
    es = jnp.exp(sim - maxs)
    lse = jnp.log(jnp.sum(es, axis=1, keepdims=True))
    m = (sim - (maxs + lse)) + (sim == maxs).astype(jnp.float32)

    out_ref[...] = lax.dot_general(
        m, cb * 0.5, (((1,), (0,)), ((), ())),
        preferred_element_type=jnp.float32)

    x2 = jnp.sum(xb * xb, axis=1, keepdims=True)
    loss_ref[...] = jnp.sum(x2 - maxs).reshape(1, 1, 1)


@jax.jit
def kernel(x, codebook):
    n = x.shape[0] * x.shape[1]
    flat_x = x.reshape(n, EMB)
    grid = n // TOKEN_BLOCK

    out, loss = pl.pallas_call(
        _vq_block,
        grid=(grid,),
        in_specs=[
            pl.BlockSpec((TOKEN_BLOCK, EMB), lambda i: (i, 0)),
            pl.BlockSpec((NUM_CODES, EMB), lambda i: (0, 0)),
        ],
        out_specs=[
            pl.BlockSpec((TOKEN_BLOCK, EMB), lambda i: (i, 0)),
            pl.BlockSpec((1, 1, 1), lambda i: (i, 0, 0)),
        ],
        out_shape=[
            jax.ShapeDtypeStruct((n, EMB), jnp.float32),
            jax.ShapeDtypeStruct((grid, 1, 1), jnp.float32),
        ],
        compiler_params=pltpu.CompilerParams(
            dimension_semantics=("parallel",)),
    )(flat_x, codebook)

    vq_loss = jnp.sum(loss) * ((1.0 + COMMIT) / (n * EMB))
    return (out.reshape(x.shape), vq_loss)
